# baseline (speedup 1.0000x reference)
import functools

import jax
import jax.numpy as jnp
from jax import lax
from jax.experimental import pallas as pl
from jax.experimental.pallas import tpu as pltpu
from jax.experimental.pallas import tpu_sc as plsc

NTOKEN = 1000000
NINP = 64
BATCH = 16384
SEQ = 50
B_TOTAL = BATCH * SEQ

_info = plsc.get_sparse_core_info()
NC = _info.num_cores
NS = _info.num_subcores
NW = NC * NS

_mesh = plsc.VectorSubcoreMesh(core_axis_name="c", subcore_axis_name="s")

NBLK_FULL = NTOKEN // 128
TAIL = NTOKEN - NBLK_FULL * 128
K1_ITERS = NBLK_FULL // NW
_IOTA = None


@functools.partial(
    pl.kernel,
    mesh=_mesh,
    out_type=jax.ShapeDtypeStruct((NTOKEN // 2, 128), jnp.float32),
    scratch_types=[
        pltpu.VMEM((64, 128), jnp.float32),
        pltpu.VMEM((64, 128), jnp.float32),
    ],
    compiler_params=pltpu.CompilerParams(
        use_tc_tiling_on_sc=True, needs_layout_passes=False),
)
def _detile(tabT_hbm, tail_hbm, r_hbm, in_v, out_v):
    w = lax.axis_index("s") * NC + lax.axis_index("c")
    iota = jnp.arange(16, dtype=jnp.int32)

    def do_block(cb):
        pltpu.sync_copy(tabT_hbm.at[:, pl.ds(cb * 128, 128)], in_v)

        def row_body(r, carry):
            for h in range(2):
                col = jnp.full((16,), 2 * r + h, jnp.int32)
                for j in range(4):
                    g = plsc.load_gather(in_v, [iota + 16 * j, col])
                    out_v[r, pl.ds(64 * h + 16 * j, 16)] = g
            return carry

        lax.fori_loop(0, 64, row_body, None)
        pltpu.sync_copy(out_v, r_hbm.at[pl.ds(cb * 64, 64), :])

    def blk_body(k, carry):
        do_block(w + k * NW)
        return carry

    lax.fori_loop(0, K1_ITERS, blk_body, None)
    @pl.when(w < NBLK_FULL - K1_ITERS * NW)
    def _():
        do_block(K1_ITERS * NW + w)

    @pl.when(w == 4)
    def _():
        pltpu.sync_copy(tail_hbm, in_v.at[pl.ds(0, 32), :])
        pltpu.sync_copy(in_v.at[pl.ds(0, 32), :],
                        r_hbm.at[pl.ds(NBLK_FULL * 64, 32), :])


NBLOCKS2 = SEQ * (BATCH // 128)
K2_PER_W = NBLOCKS2 // NW


@functools.partial(
    pl.kernel,
    mesh=_mesh,
    out_type=jax.ShapeDtypeStruct((SEQ, 8, BATCH // 128, 8, 128), jnp.float32),
    scratch_types=[
        pltpu.VMEM((128,), jnp.int32),
        pltpu.VMEM((128, NINP), jnp.float32),
        pltpu.VMEM((8, 8, 128), jnp.float32),
        pltpu.SemaphoreType.DMA,
    ],
    compiler_params=pltpu.CompilerParams(
        use_tc_tiling_on_sc=False, needs_layout_passes=False),
)
def _gather_retile(idx_hbm, r_hbm, out_hbm, idx_v, rows_v, tile_v, gsem):
    w = lax.axis_index("s") * NC + lax.axis_index("c")
    iota = jnp.arange(16, dtype=jnp.int32)

    def blk_body(t, _):
        blk = w * K2_PER_W + t
        s = blk // (BATCH // 128)
        bh = blk % (BATCH // 128)
        pltpu.sync_copy(idx_hbm.at[pl.ds(s * BATCH + bh * 128, 128)], idx_v)
        pltpu.async_copy(r_hbm.at[idx_v], rows_v, gsem).wait()

        for dh in range(8):
            def dl_body(dl, _):
                col = jnp.full((16,), 8 * dh + dl, jnp.int32)
                for j in range(8):
                    g = plsc.load_gather(rows_v, [iota + 16 * j, col])
                    tile_v[dh, dl, pl.ds(16 * j, 16)] = g
                return _

            lax.fori_loop(0, 8, dl_body, None)
        for dh in range(8):
            pltpu.sync_copy(tile_v.at[dh], out_hbm.at[s, dh, bh])
        return _

    lax.fori_loop(0, K2_PER_W, blk_body, None)


def kernel(input, table):
    tabT = jnp.transpose(table)
    tail = table[NBLK_FULL * 128:, :].reshape(32, 128)
    r2d = _detile(tabT, tail)
    rowmajor = r2d.reshape(NTOKEN, NINP)
    idx_t = jnp.transpose(input).reshape(B_TOTAL).astype(jnp.int32)
    out5 = _gather_retile(idx_t, rowmajor)
    return jnp.transpose(out5, (2, 4, 0, 1, 3)).reshape(BATCH, SEQ, NINP)


# speedup vs baseline: 1.5714x; 1.5714x over previous
import functools

import jax
import jax.numpy as jnp
from jax import lax
from jax.experimental import pallas as pl
from jax.experimental.pallas import tpu as pltpu
from jax.experimental.pallas import tpu_sc as plsc

NTOKEN = 1000000
NINP = 64
BATCH = 16384
SEQ = 50
B_TOTAL = BATCH * SEQ

_info = plsc.get_sparse_core_info()
NC = _info.num_cores
NS = _info.num_subcores
NW = NC * NS

_mesh = plsc.VectorSubcoreMesh(core_axis_name="c", subcore_axis_name="s")

NBLK_FULL = NTOKEN // 128
TAIL = NTOKEN - NBLK_FULL * 128
K1_ITERS = NBLK_FULL // NW
_IOTA = None


@functools.partial(
    pl.kernel,
    mesh=_mesh,
    out_type=jax.ShapeDtypeStruct((NTOKEN // 2, 128), jnp.float32),
    scratch_types=[
        pltpu.VMEM((64, 128), jnp.float32),
        pltpu.VMEM((64, 128), jnp.float32),
    ],
    compiler_params=pltpu.CompilerParams(
        use_tc_tiling_on_sc=True, needs_layout_passes=False),
)
def _detile(tabT_hbm, tail_hbm, r_hbm, in_v, out_v):
    w = lax.axis_index("s") * NC + lax.axis_index("c")
    iota = jnp.arange(16, dtype=jnp.int32)

    rows = [iota + 16 * j for j in range(4)]

    def do_block(cb):
        pltpu.sync_copy(tabT_hbm.at[:, pl.ds(cb * 128, 128)], in_v)

        @plsc.parallel_loop(0, 64, unroll=4)
        def row_body(r):
            for h in range(2):
                col = jnp.full((16,), 2 * r + h, jnp.int32)
                for j in range(4):
                    g = plsc.load_gather(in_v, [rows[j], col])
                    out_v[r, pl.ds(64 * h + 16 * j, 16)] = g

        pltpu.sync_copy(out_v, r_hbm.at[pl.ds(cb * 64, 64), :])

    def blk_body(k, carry):
        do_block(w + k * NW)
        return carry

    lax.fori_loop(0, K1_ITERS, blk_body, None)
    @pl.when(w < NBLK_FULL - K1_ITERS * NW)
    def _():
        do_block(K1_ITERS * NW + w)

    @pl.when(w == 4)
    def _():
        pltpu.sync_copy(tail_hbm, in_v.at[pl.ds(0, 32), :])
        pltpu.sync_copy(in_v.at[pl.ds(0, 32), :],
                        r_hbm.at[pl.ds(NBLK_FULL * 64, 32), :])


NBLOCKS2 = SEQ * (BATCH // 128)
K2_PER_W = NBLOCKS2 // NW


@functools.partial(
    pl.kernel,
    mesh=_mesh,
    out_type=jax.ShapeDtypeStruct((SEQ, 8, BATCH // 128, 8, 128), jnp.float32),
    scratch_types=[
        pltpu.VMEM((128,), jnp.int32),
        pltpu.VMEM((128, NINP), jnp.float32),
        pltpu.VMEM((NINP, 128), jnp.float32),
        pltpu.SemaphoreType.DMA,
    ],
    compiler_params=pltpu.CompilerParams(
        use_tc_tiling_on_sc=False, needs_layout_passes=False),
)
def _gather_retile(idx_hbm, r_hbm, out_hbm, idx_v, rows_v, tile_v, gsem):
    w = lax.axis_index("s") * NC + lax.axis_index("c")
    iota = jnp.arange(16, dtype=jnp.int32)

    rows = [iota + 16 * j for j in range(8)]

    def blk_body(t, carry):
        blk = w * K2_PER_W + t
        s = blk // (BATCH // 128)
        bh = blk % (BATCH // 128)
        pltpu.sync_copy(idx_hbm.at[pl.ds(s * BATCH + bh * 128, 128)], idx_v)
        pltpu.async_copy(r_hbm.at[idx_v], rows_v, gsem).wait()

        @plsc.parallel_loop(0, NINP, unroll=4)
        def d_body(d):
            col = jnp.full((16,), d, jnp.int32)
            for j in range(8):
                g = plsc.load_gather(rows_v, [rows[j], col])
                tile_v[d, pl.ds(16 * j, 16)] = g

        for dh in range(8):
            pltpu.sync_copy(tile_v.at[pl.ds(8 * dh, 8), :],
                            out_hbm.at[s, dh, bh])
        return carry

    lax.fori_loop(0, K2_PER_W, blk_body, None)


def kernel(input, table):
    tabT = jnp.transpose(table)
    tail = table[NBLK_FULL * 128:, :].reshape(32, 128)
    r2d = _detile(tabT, tail)
    rowmajor = r2d.reshape(NTOKEN, NINP)
    idx_t = jnp.transpose(input).reshape(B_TOTAL).astype(jnp.int32)
    out5 = _gather_retile(idx_t, rowmajor)
    return jnp.transpose(out5, (2, 4, 0, 1, 3)).reshape(BATCH, SEQ, NINP)


# speedup vs baseline: 2.1079x; 1.3415x over previous
import functools

import jax
import jax.numpy as jnp
from jax import lax
from jax.experimental import pallas as pl
from jax.experimental.pallas import tpu as pltpu
from jax.experimental.pallas import tpu_sc as plsc

NTOKEN = 1000000
NINP = 64
BATCH = 16384
SEQ = 50
B_TOTAL = BATCH * SEQ

_info = plsc.get_sparse_core_info()
NC = _info.num_cores
NS = _info.num_subcores
NW = NC * NS

_mesh = plsc.VectorSubcoreMesh(core_axis_name="c", subcore_axis_name="s")

NBLK_FULL = NTOKEN // 128
TAIL = NTOKEN - NBLK_FULL * 128
K1_ITERS = NBLK_FULL // NW
_IOTA = None


@functools.partial(
    pl.kernel,
    mesh=_mesh,
    out_type=jax.ShapeDtypeStruct((NTOKEN // 2, 128), jnp.float32),
    scratch_types=[
        pltpu.VMEM((64, 129), jnp.float32),
        pltpu.VMEM((64, 128), jnp.float32),
    ],
    compiler_params=pltpu.CompilerParams(
        use_tc_tiling_on_sc=True, needs_layout_passes=False),
)
def _detile(tabT_hbm, tail_hbm, r_hbm, in_v, out_v):
    w = lax.axis_index("s") * NC + lax.axis_index("c")
    iota = jnp.arange(16, dtype=jnp.int32)

    rows = [iota + 16 * j for j in range(4)]

    def do_block(cb):
        pltpu.sync_copy(tabT_hbm.at[:, pl.ds(cb * 128, 128)],
                        in_v.at[:, pl.ds(0, 128)])

        @plsc.parallel_loop(0, 64, unroll=4)
        def row_body(r):
            for h in range(2):
                col = jnp.full((16,), 2 * r + h, jnp.int32)
                for j in range(4):
                    g = plsc.load_gather(in_v, [rows[j], col])
                    out_v[r, pl.ds(64 * h + 16 * j, 16)] = g

        pltpu.sync_copy(out_v, r_hbm.at[pl.ds(cb * 64, 64), :])

    def blk_body(k, carry):
        do_block(w + k * NW)
        return carry

    lax.fori_loop(0, K1_ITERS, blk_body, None)
    @pl.when(w < NBLK_FULL - K1_ITERS * NW)
    def _():
        do_block(K1_ITERS * NW + w)

    @pl.when(w == 4)
    def _():
        pltpu.sync_copy(tail_hbm, out_v.at[pl.ds(0, 32), :])
        pltpu.sync_copy(out_v.at[pl.ds(0, 32), :],
                        r_hbm.at[pl.ds(NBLK_FULL * 64, 32), :])


NBLOCKS2 = SEQ * (BATCH // 128)
K2_PER_W = NBLOCKS2 // NW


@functools.partial(
    pl.kernel,
    mesh=_mesh,
    out_type=jax.ShapeDtypeStruct((SEQ, 8, BATCH // 128, 8, 128), jnp.float32),
    scratch_types=[
        pltpu.VMEM((128,), jnp.int32),
        pltpu.VMEM((128, NINP), jnp.float32),
        pltpu.VMEM((NINP, 129), jnp.float32),
        pltpu.SemaphoreType.DMA,
    ],
    compiler_params=pltpu.CompilerParams(
        use_tc_tiling_on_sc=False, needs_layout_passes=False),
)
def _gather_retile(idx_hbm, r_hbm, out_hbm, idx_v, rows_v, tile_v, gsem):
    w = lax.axis_index("s") * NC + lax.axis_index("c")
    iota = jnp.arange(16, dtype=jnp.int32)

    rows = [iota + 16 * j for j in range(8)]

    def blk_body(t, carry):
        blk = w * K2_PER_W + t
        s = blk // (BATCH // 128)
        bh = blk % (BATCH // 128)
        pltpu.sync_copy(idx_hbm.at[pl.ds(s * BATCH + bh * 128, 128)], idx_v)
        pltpu.async_copy(r_hbm.at[idx_v], rows_v, gsem).wait()

        @plsc.parallel_loop(0, 128, unroll=4)
        def bl_body(bl):
            col = jnp.full((16,), bl, jnp.int32)
            for j in range(4):
                x = rows_v[bl, pl.ds(16 * j, 16)]
                plsc.store_scatter(tile_v, [rows[j], col], x)

        for dh in range(8):
            pltpu.sync_copy(tile_v.at[pl.ds(8 * dh, 8), pl.ds(0, 128)],
                            out_hbm.at[s, dh, bh])
        return carry

    lax.fori_loop(0, K2_PER_W, blk_body, None)


def kernel(input, table):
    tabT = jnp.transpose(table)
    tail = table[NBLK_FULL * 128:, :].reshape(32, 128)
    r2d = _detile(tabT, tail)
    rowmajor = r2d.reshape(NTOKEN, NINP)
    idx_t = jnp.transpose(input).reshape(B_TOTAL).astype(jnp.int32)
    out5 = _gather_retile(idx_t, rowmajor)
    return jnp.transpose(out5, (2, 4, 0, 1, 3)).reshape(BATCH, SEQ, NINP)


# speedup vs baseline: 2.1097x; 1.0008x over previous
import functools

import jax
import jax.numpy as jnp
from jax import lax
from jax.experimental import pallas as pl
from jax.experimental.pallas import tpu as pltpu
from jax.experimental.pallas import tpu_sc as plsc

NTOKEN = 1000000
NINP = 64
BATCH = 16384
SEQ = 50
B_TOTAL = BATCH * SEQ

_info = plsc.get_sparse_core_info()
NC = _info.num_cores
NS = _info.num_subcores
NW = NC * NS

_mesh = plsc.VectorSubcoreMesh(core_axis_name="c", subcore_axis_name="s")

NBLK_FULL = NTOKEN // 128
TAIL = NTOKEN - NBLK_FULL * 128
K1_ITERS = NBLK_FULL // NW
_IOTA = None


@functools.partial(
    pl.kernel,
    mesh=_mesh,
    out_type=jax.ShapeDtypeStruct((NTOKEN // 2, 128), jnp.float32),
    scratch_types=[
        pltpu.VMEM((64, 136), jnp.float32),
        pltpu.VMEM((64, 128), jnp.float32),
    ],
    compiler_params=pltpu.CompilerParams(
        use_tc_tiling_on_sc=True, needs_layout_passes=False),
)
def _detile(tabT_hbm, tail_hbm, r_hbm, in_v, out_v):
    w = lax.axis_index("s") * NC + lax.axis_index("c")
    iota = jnp.arange(16, dtype=jnp.int32)

    rows = [iota + 16 * j for j in range(4)]

    def do_block(cb):
        pltpu.sync_copy(tabT_hbm.at[:, pl.ds(cb * 128, 128)],
                        in_v.at[:, pl.ds(0, 128)])

        @plsc.parallel_loop(0, 64, unroll=4)
        def row_body(r):
            for h in range(2):
                col = jnp.full((16,), 2 * r + h, jnp.int32)
                for j in range(4):
                    g = plsc.load_gather(in_v, [rows[j], col])
                    out_v[r, pl.ds(64 * h + 16 * j, 16)] = g

        pltpu.sync_copy(out_v, r_hbm.at[pl.ds(cb * 64, 64), :])

    def blk_body(k, carry):
        do_block(w + k * NW)
        return carry

    lax.fori_loop(0, K1_ITERS, blk_body, None)
    @pl.when(w < NBLK_FULL - K1_ITERS * NW)
    def _():
        do_block(K1_ITERS * NW + w)

    @pl.when(w == 4)
    def _():
        pltpu.sync_copy(tail_hbm, out_v.at[pl.ds(0, 32), :])
        pltpu.sync_copy(out_v.at[pl.ds(0, 32), :],
                        r_hbm.at[pl.ds(NBLK_FULL * 64, 32), :])


NBLOCKS2 = SEQ * (BATCH // 128)
K2_PER_W = NBLOCKS2 // NW


@functools.partial(
    pl.kernel,
    mesh=_mesh,
    out_type=jax.ShapeDtypeStruct((SEQ, 8, BATCH // 128, 8, 128), jnp.float32),
    scratch_types=[
        pltpu.VMEM((128,), jnp.int32),
        pltpu.VMEM((128, NINP), jnp.float32),
        pltpu.VMEM((NINP, 129), jnp.float32),
        pltpu.SemaphoreType.DMA,
    ],
    compiler_params=pltpu.CompilerParams(
        use_tc_tiling_on_sc=False, needs_layout_passes=False),
)
def _gather_retile(idx_hbm, r_hbm, out_hbm, idx_v, rows_v, tile_v, gsem):
    w = lax.axis_index("s") * NC + lax.axis_index("c")
    iota = jnp.arange(16, dtype=jnp.int32)

    rows = [iota + 16 * j for j in range(8)]

    def blk_body(t, carry):
        blk = w * K2_PER_W + t
        s = blk // (BATCH // 128)
        bh = blk % (BATCH // 128)
        pltpu.sync_copy(idx_hbm.at[pl.ds(s * BATCH + bh * 128, 128)], idx_v)
        pltpu.async_copy(r_hbm.at[idx_v], rows_v, gsem).wait()

        @plsc.parallel_loop(0, 128, unroll=4)
        def bl_body(bl):
            col = jnp.full((16,), bl, jnp.int32)
            for j in range(4):
                x = rows_v[bl, pl.ds(16 * j, 16)]
                plsc.store_scatter(tile_v, [rows[j], col], x)

        for dh in range(8):
            pltpu.sync_copy(tile_v.at[pl.ds(8 * dh, 8), pl.ds(0, 128)],
                            out_hbm.at[s, dh, bh])
        return carry

    lax.fori_loop(0, K2_PER_W, blk_body, None)


def kernel(input, table):
    tabT = jnp.transpose(table)
    tail = table[NBLK_FULL * 128:, :].reshape(32, 128)
    r2d = _detile(tabT, tail)
    rowmajor = r2d.reshape(NTOKEN, NINP)
    idx_t = jnp.transpose(input).reshape(B_TOTAL).astype(jnp.int32)
    out5 = _gather_retile(idx_t, rowmajor)
    return jnp.transpose(out5, (2, 4, 0, 1, 3)).reshape(BATCH, SEQ, NINP)


# speedup vs baseline: 3.0753x; 1.4577x over previous
import functools

import jax
import jax.numpy as jnp
from jax import lax
from jax.experimental import pallas as pl
from jax.experimental.pallas import tpu as pltpu
from jax.experimental.pallas import tpu_sc as plsc

NTOKEN = 1000000
NINP = 64
BATCH = 16384
SEQ = 50
B_TOTAL = BATCH * SEQ

_info = plsc.get_sparse_core_info()
NC = _info.num_cores
NS = _info.num_subcores
NW = NC * NS

_mesh = plsc.VectorSubcoreMesh(core_axis_name="c", subcore_axis_name="s")

NBLK_FULL = NTOKEN // 128
TAIL = NTOKEN - NBLK_FULL * 128
K1_ITERS = NBLK_FULL // NW
_IOTA = None


@functools.partial(
    pl.kernel,
    mesh=_mesh,
    out_type=jax.ShapeDtypeStruct((NTOKEN // 2, 128), jnp.float32),
    scratch_types=[
        pltpu.VMEM((64, 136), jnp.float32),
        pltpu.VMEM((64, 136), jnp.float32),
        pltpu.VMEM((64, 128), jnp.float32),
        pltpu.VMEM((64, 128), jnp.float32),
        pltpu.SemaphoreType.DMA,
        pltpu.SemaphoreType.DMA,
        pltpu.SemaphoreType.DMA,
        pltpu.SemaphoreType.DMA,
    ],
    compiler_params=pltpu.CompilerParams(
        use_tc_tiling_on_sc=True, needs_layout_passes=False),
)
def _detile(tabT_hbm, tail_hbm, r_hbm, in_a, in_b, out_a, out_b,
            si_a, si_b, so_a, so_b):
    w = lax.axis_index("s") * NC + lax.axis_index("c")
    iota = jnp.arange(16, dtype=jnp.int32)

    rows = [iota + 16 * j for j in range(4)]
    ins = (in_a, in_b)
    outs = (out_a, out_b)
    sis = (si_a, si_b)
    sos = (so_a, so_b)

    def in_desc(cb, p):
        return pltpu.make_async_copy(
            tabT_hbm.at[:, pl.ds(cb * 128, 128)],
            ins[p].at[:, pl.ds(0, 128)], sis[p])

    def out_desc(cb, p):
        return pltpu.make_async_copy(
            outs[p], r_hbm.at[pl.ds(cb * 64, 64), :], sos[p])

    def transpose(p):
        in_v, out_v = ins[p], outs[p]

        @plsc.parallel_loop(0, 64, unroll=4)
        def row_body(r):
            for h in range(2):
                col = jnp.full((16,), 2 * r + h, jnp.int32)
                for j in range(4):
                    g = plsc.load_gather(in_v, [rows[j], col])
                    out_v[r, pl.ds(64 * h + 16 * j, 16)] = g

    in_desc(w, 0).start()

    def blk_body(i, carry):
        for p in range(2):
            k = 2 * i + p
            cb = w + k * NW
            in_desc(cb, p).wait()
            @pl.when(k + 1 < K1_ITERS)
            def _():
                in_desc(cb + NW, 1 - p).start()

            @pl.when(k >= 2)
            def _():
                out_desc(cb - 2 * NW, p).wait()

            transpose(p)
            out_desc(cb, p).start()
        return carry

    lax.fori_loop(0, K1_ITERS // 2, blk_body, None)
    out_desc(w + (K1_ITERS - 2) * NW, 0).wait()
    out_desc(w + (K1_ITERS - 1) * NW, 1).wait()
    @pl.when(w < NBLK_FULL - K1_ITERS * NW)
    def _():
        cb = K1_ITERS * NW + w
        in_desc(cb, 0).start()
        in_desc(cb, 0).wait()
        transpose(0)
        out_desc(cb, 0).start()
        out_desc(cb, 0).wait()

    @pl.when(w == 4)
    def _():
        pltpu.sync_copy(tail_hbm, out_b.at[pl.ds(0, 32), :])
        pltpu.sync_copy(out_b.at[pl.ds(0, 32), :],
                        r_hbm.at[pl.ds(NBLK_FULL * 64, 32), :])


NBLOCKS2 = SEQ * (BATCH // 128)
K2_PER_W = NBLOCKS2 // NW


@functools.partial(
    pl.kernel,
    mesh=_mesh,
    out_type=jax.ShapeDtypeStruct((SEQ, 8, BATCH // 128, 8, 128), jnp.float32),
    scratch_types=[
        pltpu.VMEM((128,), jnp.int32),
        pltpu.VMEM((128,), jnp.int32),
        pltpu.VMEM((128, NINP), jnp.float32),
        pltpu.VMEM((128, NINP), jnp.float32),
        pltpu.VMEM((NINP, 129), jnp.float32),
        pltpu.VMEM((NINP, 129), jnp.float32),
        pltpu.SemaphoreType.DMA,
        pltpu.SemaphoreType.DMA,
        pltpu.SemaphoreType.DMA,
        pltpu.SemaphoreType.DMA,
        pltpu.SemaphoreType.DMA,
        pltpu.SemaphoreType.DMA,
    ],
    compiler_params=pltpu.CompilerParams(
        use_tc_tiling_on_sc=False, needs_layout_passes=False),
)
def _gather_retile(idx_hbm, r_hbm, out_hbm, idx_a, idx_b, rows_a, rows_b,
                   tile_a, tile_b, sx_a, sx_b, sg_a, sg_b, so_a, so_b):
    w = lax.axis_index("s") * NC + lax.axis_index("c")
    iota = jnp.arange(16, dtype=jnp.int32)

    rows = [iota + 16 * j for j in range(4)]
    idxs = (idx_a, idx_b)
    rowss = (rows_a, rows_b)
    tiles = (tile_a, tile_b)
    sxs = (sx_a, sx_b)
    sgs = (sg_a, sg_b)
    sos = (so_a, so_b)

    def sbh(t):
        blk = w * K2_PER_W + t
        return blk // (BATCH // 128), blk % (BATCH // 128)

    def idx_desc(t, p):
        s, bh = sbh(t)
        return pltpu.make_async_copy(
            idx_hbm.at[pl.ds(s * BATCH + bh * 128, 128)], idxs[p], sxs[p])

    def out_desc(t, p, dh):
        s, bh = sbh(t)
        return pltpu.make_async_copy(
            tiles[p].at[pl.ds(8 * dh, 8), pl.ds(0, 128)],
            out_hbm.at[s, dh, bh], sos[p])

    def transpose(p):
        rows_v, tile_v = rowss[p], tiles[p]

        @plsc.parallel_loop(0, 128, unroll=4)
        def bl_body(bl):
            col = jnp.full((16,), bl, jnp.int32)
            for j in range(4):
                x = rows_v[bl, pl.ds(16 * j, 16)]
                plsc.store_scatter(tile_v, [rows[j], col], x)

    idx_desc(0, 0).start()

    def blk_body(i, carry):
        for p in range(2):
            t = 2 * i + p
            idx_desc(t, p).wait()
            pltpu.async_copy(r_hbm.at[idxs[p]], rowss[p], sgs[p])
            @pl.when(t + 1 < K2_PER_W)
            def _():
                idx_desc(t + 1, 1 - p).start()

            pltpu.make_async_copy(r_hbm.at[idxs[p]], rowss[p], sgs[p]).wait()

            @pl.when(t >= 2)
            def _():
                for dh in range(8):
                    out_desc(t - 2, p, dh).wait()

            transpose(p)
            for dh in range(8):
                out_desc(t, p, dh).start()
        return carry

    lax.fori_loop(0, K2_PER_W // 2, blk_body, None)
    for p in range(2):
        for dh in range(8):
            out_desc(K2_PER_W - 2 + p, p, dh).wait()


def kernel(input, table):
    tabT = jnp.transpose(table)
    tail = table[NBLK_FULL * 128:, :].reshape(32, 128)
    r2d = _detile(tabT, tail)
    rowmajor = r2d.reshape(NTOKEN, NINP)
    idx_t = jnp.transpose(input).reshape(B_TOTAL).astype(jnp.int32)
    out5 = _gather_retile(idx_t, rowmajor)
    return jnp.transpose(out5, (2, 4, 0, 1, 3)).reshape(BATCH, SEQ, NINP)
